# Initial kernel scaffold; baseline (speedup 1.0000x reference)
import jax
import jax.numpy as jnp
from jax.experimental import pallas as pl


def kernel(x, w1, w2):
    raise NotImplementedError("write your pallas kernel here")


# speedup vs baseline: 1.1396x; 1.1396x over previous
import functools

import jax
import jax.numpy as jnp
from jax import lax
from jax.experimental import pallas as pl
from jax.experimental.pallas import tpu as pltpu

_MIB = 1 << 20


def _se_kernel(x_ref, w1_ref, w2_ref, o_ref, *, inv_hw):
    x = x_ref[...]
    pooled = jnp.sum(x, axis=-1, dtype=jnp.float32) * inv_hw
    h = lax.dot_general(pooled, w1_ref[...], (((1,), (1,)), ((), ())),
                        preferred_element_type=jnp.float32)
    h = jnp.maximum(h, 0.0)
    s = lax.dot_general(h, w2_ref[...], (((1,), (1,)), ((), ())),
                        preferred_element_type=jnp.float32)
    s = jax.nn.sigmoid(s)
    o_ref[...] = x * s.astype(x.dtype)[:, :, None]


def kernel(x, w1, w2):
    B, C, H, W = x.shape
    HW = H * W
    hidden = w1.shape[0]
    inv_hw = 1.0 / float(HW)

    x3 = x.reshape(B, C, HW)

    bt = 1
    block_bytes = bt * C * HW * x.dtype.itemsize
    vmem_limit = int(min(63 * _MIB, 4 * block_bytes + 8 * _MIB))

    out3 = pl.pallas_call(
        functools.partial(_se_kernel, inv_hw=inv_hw),
        out_shape=jax.ShapeDtypeStruct((B, C, HW), x.dtype),
        grid=(B // bt,),
        in_specs=[
            pl.BlockSpec((bt, C, HW), lambda b: (b, 0, 0)),
            pl.BlockSpec((hidden, C), lambda b: (0, 0)),
            pl.BlockSpec((C, hidden), lambda b: (0, 0)),
        ],
        out_specs=pl.BlockSpec((bt, C, HW), lambda b: (b, 0, 0)),
        compiler_params=pltpu.CompilerParams(
            dimension_semantics=("parallel",),
            vmem_limit_bytes=vmem_limit,
        ),
    )(x3, w1, w2)

    return out3.reshape(B, C, H, W)
